# baseline (speedup 1.0000x reference)
import jax
import jax.numpy as jnp
from jax.experimental import pallas as pl
from jax.experimental.pallas import tpu as pltpu

_N = 1048576
_ROWS = 1024
_COLS = 1024
_G = 8
_BLK = _ROWS // _G
_EFFICIENCY = 0.995
_NUM_PARENTS = 2.0
_INV_LOSS = float(1.0 / (_EFFICIENCY**_NUM_PARENTS))


def _body(x_hbm, o_ref, vbuf, acc_ref, in_sems):
    i = pl.program_id(0)

    def in_copy(k):
        return pltpu.make_async_copy(
            x_hbm.at[pl.ds(k * _BLK, _BLK), :], vbuf.at[k], in_sems.at[k]
        )

    @pl.when(i == 0)
    def _start_all():
        for k in range(_G):
            in_copy(k).start()
        acc_ref[0] = 0.0

    @pl.when(i < _G)
    def _reduce():
        pltpu.make_async_copy(
            x_hbm.at[pl.ds(i * _BLK, _BLK), :],
            vbuf.at[i],
            in_sems.at[i],
        ).wait()
        acc_ref[0] += jnp.sum(vbuf[i])

    @pl.when(i >= _G)
    def _fill():
        o_ref[...] = jnp.full((_BLK, _COLS), acc_ref[0] * _INV_LOSS, jnp.float32)


def kernel(charger_rate_current, charger_idx):
    del charger_idx
    x = charger_rate_current.reshape(_ROWS, _COLS)
    out = pl.pallas_call(
        _body,
        grid=(2 * _G,),
        in_specs=[pl.BlockSpec(memory_space=pl.ANY)],
        out_specs=pl.BlockSpec((_BLK, _COLS), lambda i: (jnp.maximum(i - _G, 0), 0)),
        out_shape=jax.ShapeDtypeStruct((_ROWS, _COLS), jnp.float32),
        scratch_shapes=[
            pltpu.VMEM((_G, _BLK, _COLS), jnp.float32),
            pltpu.SMEM((1,), jnp.float32),
            pltpu.SemaphoreType.DMA((_G,)),
        ],
    )(x)
    return out.reshape(_N)


# speedup vs baseline: 1.4506x; 1.4506x over previous
import jax
import jax.numpy as jnp
from jax.experimental import pallas as pl
from jax.experimental.pallas import tpu as pltpu

_N = 1048576
_ROWS = 1024
_COLS = 1024
_G = 8
_BLK = _ROWS // _G
_EFFICIENCY = 0.995
_NUM_PARENTS = 2.0
_INV_LOSS = float(1.0 / (_EFFICIENCY**_NUM_PARENTS))


def _body(x_hbm, o_ref, vbuf, sems):
    def in_copy(i):
        return pltpu.make_async_copy(
            x_hbm.at[pl.ds(i * _BLK, _BLK), :], vbuf.at[i], sems.at[i]
        )

    for i in range(_G):
        in_copy(i).start()
    acc = jnp.float32(0.0)
    for i in range(_G):
        in_copy(i).wait()
        acc = acc + jnp.sum(vbuf[i])
    o_ref[0] = acc * _INV_LOSS


def kernel(charger_rate_current, charger_idx):
    del charger_idx
    x = charger_rate_current.reshape(_ROWS, _COLS)
    total = pl.pallas_call(
        _body,
        in_specs=[pl.BlockSpec(memory_space=pl.ANY)],
        out_specs=pl.BlockSpec(memory_space=pltpu.SMEM),
        out_shape=jax.ShapeDtypeStruct((1,), jnp.float32),
        scratch_shapes=[
            pltpu.VMEM((_G, _BLK, _COLS), jnp.float32),
            pltpu.SemaphoreType.DMA((_G,)),
        ],
    )(x)
    return jnp.broadcast_to(total, (_N,))
